# Initial kernel scaffold; baseline (speedup 1.0000x reference)
import jax
import jax.numpy as jnp
from jax.experimental import pallas as pl


def kernel(x, adc_char):
    raise NotImplementedError("write your pallas kernel here")


# speedup vs baseline: 4374.6434x; 4374.6434x over previous
import jax
import jax.numpy as jnp
from jax.experimental import pallas as pl
from jax.experimental.pallas import tpu as pltpu

_BLOCK_ROWS = 256
_SUB = _BLOCK_ROWS // 8
_GROUP = 8


def _tree_select(masks, vals):
    cur = list(vals)
    for m in reversed(masks):
        cur = [jnp.where(m, cur[2 * i + 1], cur[2 * i]) for i in range(len(cur) // 2)]
    return cur[0]


def _adc_body(scal_ref, tab_ref, x_ref, o_ref):
    sc = [scal_ref[i] for i in range(31)]
    lvl = [sc[0:1], sc[1:3], sc[3:7], sc[7:15], sc[15:31]]
    tabs = [tab_ref[8 * k:8 * k + 8, :] for k in range(3)]

    for g in range(_SUB // _GROUP):
        base = g * _GROUP
        for j in range(_GROUP):
            r = base + j
            x = x_ref[8 * r:8 * r + 8, :]
            masks = []
            for k in range(5):
                thr = lvl[k][0] if k == 0 else _tree_select(masks, lvl[k])
                masks.append(thr <= x)
            n = jnp.where(masks[0], 16, 0)
            n = n + jnp.where(masks[1], 8, 0)
            n = n + jnp.where(masks[2], 4, 0)
            n = n + jnp.where(masks[3], 2, 0)
            n = n + jnp.where(masks[4], 1, 0)
            for t in range(3):
                thr = jnp.take_along_axis(
                    tabs[t], n, axis=1, mode="promise_in_bounds")
                c = (thr <= x).astype(jnp.int32)
                n = n + n + c
            o_ref[8 * r:8 * r + 8, :] = (
                n.astype(jnp.float32) * jnp.float32(1.0 / 128.0) - jnp.float32(1.0)
            )


def kernel(x, adc_char):
    orig_shape = x.shape
    n_rows = x.size // 1024
    x2 = x.reshape(n_rows, 1024)

    def level_vals(k):
        j = jnp.arange(1 << k)
        return adc_char[(2 * j + 1) * (1 << (7 - k)) - 1]

    scal = jnp.concatenate([level_vals(k) for k in range(5)])
    scal = jnp.pad(scal, (0, 1)).astype(jnp.float32)

    j = jnp.arange(128)
    rows = []
    for k in (5, 6, 7):
        idx = (2 * j + 1) * (1 << (7 - k)) - 1
        rows.append(jnp.where(j < (1 << k), adc_char[jnp.clip(idx, 0, 254)], 0.0))
    tables = jnp.repeat(jnp.stack(rows).astype(jnp.float32), 8, axis=0)

    grid = (n_rows // _BLOCK_ROWS,)
    out = pl.pallas_call(
        _adc_body,
        grid=grid,
        in_specs=[
            pl.BlockSpec(memory_space=pltpu.SMEM),
            pl.BlockSpec((24, 128), lambda i: (0, 0)),
            pl.BlockSpec((_BLOCK_ROWS, 1024), lambda i: (i, 0)),
        ],
        out_specs=pl.BlockSpec((_BLOCK_ROWS, 1024), lambda i: (i, 0)),
        out_shape=jax.ShapeDtypeStruct((n_rows, 1024), jnp.float32),
        compiler_params=pltpu.CompilerParams(
            dimension_semantics=("parallel",),
        ),
    )(scal, tables, x2)
    return out.reshape(orig_shape)
